# baseline (device time: 22385 ns/iter reference)
import jax
import jax.numpy as jnp
from jax import lax
from jax.experimental import pallas as pl
from jax.experimental.pallas import tpu as pltpu

C = 16


def kernel(x):
    m, n = x.shape
    half = m // 2
    chunk = half // C

    def body(x_ref, out_ref, comm_ref, xs, xr, ys, yr, own_sem, cp_sems):
        my_x = lax.axis_index("x")
        my_y = lax.axis_index("y")
        other_x = 1 - my_x
        other_y = 1 - my_y

        barrier = pltpu.get_barrier_semaphore()
        for nbr in ((other_x, my_y), (my_x, other_y)):
            pl.semaphore_signal(
                barrier, inc=1, device_id=nbr,
                device_id_type=pl.DeviceIdType.MESH,
            )
        pl.semaphore_wait(barrier, 2)

        own = pltpu.make_async_copy(
            x_ref, out_ref.at[pl.ds(my_x * m, m), :], own_sem
        )
        own.start()

        x_rdmas = []
        for c in range(C):
            r = pltpu.make_async_remote_copy(
                src_ref=x_ref.at[pl.ds(my_y * half + c * chunk, chunk), :],
                dst_ref=comm_ref.at[pl.ds(c * chunk, chunk), :],
                send_sem=xs.at[c],
                recv_sem=xr.at[c],
                device_id=(other_x, my_y),
                device_id_type=pl.DeviceIdType.MESH,
            )
            r.start()
            x_rdmas.append(r)

        y_rdmas = []
        copies = []
        for c in range(C):
            src = comm_ref.at[pl.ds(c * chunk, chunk), :]
            recv = pltpu.make_async_remote_copy(
                src_ref=src,
                dst_ref=src,
                send_sem=xs.at[c],
                recv_sem=xr.at[c],
                device_id=(other_x, my_y),
                device_id_type=pl.DeviceIdType.MESH,
            )
            recv.wait_recv()
            dst_off = other_x * m + my_y * half + c * chunk
            r = pltpu.make_async_remote_copy(
                src_ref=src,
                dst_ref=out_ref.at[pl.ds(dst_off, chunk), :],
                send_sem=ys.at[c],
                recv_sem=yr.at[c],
                device_id=(my_x, other_y),
                device_id_type=pl.DeviceIdType.MESH,
            )
            r.start()
            y_rdmas.append(r)
            cp = pltpu.make_async_copy(
                src, out_ref.at[pl.ds(dst_off, chunk), :], cp_sems.at[c]
            )
            cp.start()
            copies.append(cp)

        for c in range(C):
            dst_off = other_x * m + other_y * half + c * chunk
            recv = pltpu.make_async_remote_copy(
                src_ref=comm_ref.at[pl.ds(c * chunk, chunk), :],
                dst_ref=out_ref.at[pl.ds(dst_off, chunk), :],
                send_sem=ys.at[c],
                recv_sem=yr.at[c],
                device_id=(my_x, other_y),
                device_id_type=pl.DeviceIdType.MESH,
            )
            recv.wait_recv()

        own.wait()
        for c in range(C):
            copies[c].wait()
            x_rdmas[c].wait_send()
            y_rdmas[c].wait_send()

    return pl.pallas_call(
        body,
        out_shape=jax.ShapeDtypeStruct((2 * m, n), x.dtype),
        in_specs=[pl.BlockSpec(memory_space=pltpu.VMEM)],
        out_specs=pl.BlockSpec(memory_space=pltpu.HBM),
        scratch_shapes=[
            pltpu.VMEM((half, n), x.dtype),
            pltpu.SemaphoreType.DMA((C,)),
            pltpu.SemaphoreType.DMA((C,)),
            pltpu.SemaphoreType.DMA((C,)),
            pltpu.SemaphoreType.DMA((C,)),
            pltpu.SemaphoreType.DMA,
            pltpu.SemaphoreType.DMA((C,)),
        ],
        compiler_params=pltpu.CompilerParams(collective_id=0),
    )(x)


# device time: 22058 ns/iter; 1.0148x vs baseline; 1.0148x over previous
import jax
import jax.numpy as jnp
from jax import lax
from jax.experimental import pallas as pl
from jax.experimental.pallas import tpu as pltpu

C = 16


def kernel(x):
    m, n = x.shape
    half = m // 2
    chunk = half // C

    def body(x_ref, out_ref, xs, xr, ys, yr):
        my_x = lax.axis_index("x")
        my_y = lax.axis_index("y")
        other_x = 1 - my_x
        other_y = 1 - my_y

        barrier = pltpu.get_barrier_semaphore()
        for nbr in ((other_x, my_y), (my_x, other_y)):
            pl.semaphore_signal(
                barrier, inc=1, device_id=nbr,
                device_id_type=pl.DeviceIdType.MESH,
            )
        pl.semaphore_wait(barrier, 2)

        x_rdmas = []
        for c in range(C):
            r = pltpu.make_async_remote_copy(
                src_ref=x_ref.at[pl.ds(my_y * half + c * chunk, chunk), :],
                dst_ref=out_ref.at[pl.ds(my_x * m + my_y * half + c * chunk, chunk), :],
                send_sem=xs.at[c],
                recv_sem=xr.at[c],
                device_id=(other_x, my_y),
                device_id_type=pl.DeviceIdType.MESH,
            )
            r.start()
            x_rdmas.append(r)

        y_rdmas = []
        for c in range(C):
            x_rdmas[c].wait_recv()
            off = other_x * m + my_y * half + c * chunk
            r = pltpu.make_async_remote_copy(
                src_ref=out_ref.at[pl.ds(off, chunk), :],
                dst_ref=out_ref.at[pl.ds(off, chunk), :],
                send_sem=ys.at[c],
                recv_sem=yr.at[c],
                device_id=(my_x, other_y),
                device_id_type=pl.DeviceIdType.MESH,
            )
            r.start()
            y_rdmas.append(r)

        out_ref[pl.ds(my_x * m, m), :] = x_ref[:, :]

        for c in range(C):
            y_rdmas[c].wait_recv()
        for c in range(C):
            x_rdmas[c].wait_send()
            y_rdmas[c].wait_send()

    return pl.pallas_call(
        body,
        out_shape=jax.ShapeDtypeStruct((2 * m, n), x.dtype),
        in_specs=[pl.BlockSpec(memory_space=pltpu.VMEM)],
        out_specs=pl.BlockSpec(memory_space=pltpu.VMEM),
        scratch_shapes=[
            pltpu.SemaphoreType.DMA((C,)),
            pltpu.SemaphoreType.DMA((C,)),
            pltpu.SemaphoreType.DMA((C,)),
            pltpu.SemaphoreType.DMA((C,)),
        ],
        compiler_params=pltpu.CompilerParams(collective_id=0),
    )(x)


# device time: 22044 ns/iter; 1.0155x vs baseline; 1.0006x over previous
import jax
import jax.numpy as jnp
from jax import lax
from jax.experimental import pallas as pl
from jax.experimental.pallas import tpu as pltpu

C = 16


def kernel(x):
    m, n = x.shape
    half = m // 2
    chunk = half // C

    def body(x_ref, out_ref, xs, xr, ys, yr, own_sem):
        my_x = lax.axis_index("x")
        my_y = lax.axis_index("y")
        other_x = 1 - my_x
        other_y = 1 - my_y

        barrier = pltpu.get_barrier_semaphore()
        for nbr in ((other_x, my_y), (my_x, other_y)):
            pl.semaphore_signal(
                barrier, inc=1, device_id=nbr,
                device_id_type=pl.DeviceIdType.MESH,
            )
        pl.semaphore_wait(barrier, 2)

        x_rdmas = []
        for c in range(C):
            r = pltpu.make_async_remote_copy(
                src_ref=x_ref.at[pl.ds(my_y * half + c * chunk, chunk), :],
                dst_ref=out_ref.at[pl.ds(my_x * m + my_y * half + c * chunk, chunk), :],
                send_sem=xs.at[c],
                recv_sem=xr.at[c],
                device_id=(other_x, my_y),
                device_id_type=pl.DeviceIdType.MESH,
            )
            r.start()
            x_rdmas.append(r)

        y_rdmas = []
        for c in range(C):
            x_rdmas[c].wait_recv()
            off = other_x * m + my_y * half + c * chunk
            r = pltpu.make_async_remote_copy(
                src_ref=out_ref.at[pl.ds(off, chunk), :],
                dst_ref=out_ref.at[pl.ds(off, chunk), :],
                send_sem=ys.at[c],
                recv_sem=yr.at[c],
                device_id=(my_x, other_y),
                device_id_type=pl.DeviceIdType.MESH,
            )
            r.start()
            y_rdmas.append(r)

        own = pltpu.make_async_copy(
            x_ref, out_ref.at[pl.ds(my_x * m, m), :], own_sem
        )
        own.start()

        for c in range(C):
            y_rdmas[c].wait_recv()
        own.wait()
        for c in range(C):
            x_rdmas[c].wait_send()
            y_rdmas[c].wait_send()

    return pl.pallas_call(
        body,
        out_shape=jax.ShapeDtypeStruct((2 * m, n), x.dtype),
        in_specs=[pl.BlockSpec(memory_space=pltpu.HBM)],
        out_specs=pl.BlockSpec(memory_space=pltpu.VMEM),
        scratch_shapes=[
            pltpu.SemaphoreType.DMA((C,)),
            pltpu.SemaphoreType.DMA((C,)),
            pltpu.SemaphoreType.DMA((C,)),
            pltpu.SemaphoreType.DMA((C,)),
            pltpu.SemaphoreType.DMA,
        ],
        compiler_params=pltpu.CompilerParams(collective_id=0),
    )(x)
